# baseline (speedup 1.0000x reference)
import functools

import jax
import jax.numpy as jnp
from jax import lax
from jax.experimental import pallas as pl
from jax.experimental.pallas import tpu as pltpu
from jax.experimental.pallas import tpu_sc as plsc

N_NODES = 10000
N_EDGES = 320000
D_FEAT = 128

NC = 2
NS = 16
NW = NC * NS
CHUNK = 40
EDGES_PER_W = N_EDGES // NW
CHUNKS_PER_W = EDGES_PER_W // CHUNK
N_PAD = 10240
ROWS_PER_TILE = N_PAD // NS
NB = 5

_mesh = plsc.VectorSubcoreMesh(core_axis_name="c", subcore_axis_name="s")


@functools.partial(
    pl.kernel,
    mesh=_mesh,
    compiler_params=pltpu.CompilerParams(use_tc_tiling_on_sc=False),
    out_type=jax.ShapeDtypeStruct((NC, N_PAD, D_FEAT), jnp.float32),
    scratch_types=[
        pltpu.VMEM((CHUNKS_PER_W, CHUNK), jnp.int32),
        pltpu.VMEM((CHUNKS_PER_W, CHUNK), jnp.int32),
        [pltpu.VMEM((CHUNK, D_FEAT), jnp.float32) for _ in range(NB)],
        pltpu.VMEM_SHARED((N_PAD, D_FEAT), jnp.float32),
        [pltpu.SemaphoreType.DMA for _ in range(NB)],
        [pltpu.SemaphoreType.DMA for _ in range(NB)],
    ],
)
def _sc_aggregate(nodes_hbm, edges_hbm, out_hbm,
                  src_v, dst_v, bufs, acc, gsems, ssems):
    c = lax.axis_index("c")
    s = lax.axis_index("s")
    wid = c * NS + s
    row0 = s * ROWS_PER_TILE

    @pl.loop(0, CHUNK)
    def _zrow(r):
        for c16 in range(D_FEAT // 16):
            bufs[0][r, pl.ds(c16 * 16, 16)] = jnp.zeros((16,), jnp.float32)
    for k in range(ROWS_PER_TILE // CHUNK):
        pltpu.sync_copy(bufs[0], acc.at[pl.ds(row0 + k * CHUNK, CHUNK)])
    chunk0 = wid * CHUNKS_PER_W
    pltpu.sync_copy(edges_hbm.at[0, pl.ds(chunk0, CHUNKS_PER_W)], src_v)
    pltpu.sync_copy(edges_hbm.at[1, pl.ds(chunk0, CHUNKS_PER_W)], dst_v)
    plsc.subcore_barrier()

    def gather(j, b):
        pltpu.async_copy(nodes_hbm.at[src_v.at[j]], bufs[b], gsems[b])

    def wait_gather(j, b):
        pltpu.make_async_copy(nodes_hbm.at[src_v.at[j]],
                              bufs[b], gsems[b]).wait()

    def scatter(j, b):
        pltpu.async_copy(bufs[b], acc.at[dst_v.at[j]], ssems[b], add=True)

    def wait_scatter(j, b):
        pltpu.make_async_copy(bufs[b], acc.at[dst_v.at[j]], ssems[b]).wait()

    for b in range(2):
        gather(b, b)

    @pl.loop(0, CHUNKS_PER_W, step=NB)
    def _(g):
        for b in range(NB):
            j = g + b
            b2 = (b + 2) % NB
            wait_gather(j, b)
            scatter(j, b)

            @pl.when(j < 3)
            def _fill():
                gather(j + 2, b2)

            @pl.when((j >= 3) & (j + 2 < CHUNKS_PER_W))
            def _recycle():
                wait_scatter(j - 3, b2)
                gather(j + 2, b2)

            @pl.when((j >= 3) & (j + 2 >= CHUNKS_PER_W))
            def _drain():
                wait_scatter(j - 3, b2)

    for t in range(CHUNKS_PER_W - 3, CHUNKS_PER_W):
        wait_scatter(t, t % NB)

    plsc.subcore_barrier()
    pltpu.sync_copy(acc.at[pl.ds(row0, ROWS_PER_TILE)],
                    out_hbm.at[c, pl.ds(row0, ROWS_PER_TILE)])


def _combine_body(p_ref, o_ref):
    o_ref[...] = p_ref[0] + p_ref[1]


_ROWS_BLK = 1000


def _combine(partials):
    return pl.pallas_call(
        _combine_body,
        grid=(N_NODES // _ROWS_BLK,),
        in_specs=[pl.BlockSpec((NC, _ROWS_BLK, D_FEAT), lambda i: (0, i, 0))],
        out_specs=pl.BlockSpec((_ROWS_BLK, D_FEAT), lambda i: (i, 0)),
        out_shape=jax.ShapeDtypeStruct((N_NODES, D_FEAT), jnp.float32),
    )(partials)


def kernel(node_features, edge_index):
    edges = edge_index.astype(jnp.int32).reshape(2, N_EDGES // CHUNK, CHUNK)
    partials = _sc_aggregate(node_features, edges)
    return _combine(partials)


# speedup vs baseline: 1.5017x; 1.5017x over previous
import functools

import jax
import jax.numpy as jnp
from jax import lax
from jax.experimental import pallas as pl
from jax.experimental.pallas import tpu as pltpu
from jax.experimental.pallas import tpu_sc as plsc

N_NODES = 10000
N_EDGES = 320000
D_FEAT = 128

NC = 2
NS = 16
NW = NC * NS
CHUNK = 80
EDGES_PER_W = N_EDGES // NW
CHUNKS_PER_W = EDGES_PER_W // CHUNK
N_PAD = 10240
ROWS_PER_TILE = N_PAD // NS
NB = 5

_mesh = plsc.VectorSubcoreMesh(core_axis_name="c", subcore_axis_name="s")


@functools.partial(
    pl.kernel,
    mesh=_mesh,
    compiler_params=pltpu.CompilerParams(use_tc_tiling_on_sc=False),
    out_type=jax.ShapeDtypeStruct((NC, N_PAD, D_FEAT), jnp.bfloat16),
    scratch_types=[
        pltpu.VMEM((CHUNKS_PER_W, CHUNK), jnp.int32),
        pltpu.VMEM((CHUNKS_PER_W, CHUNK), jnp.int32),
        [pltpu.VMEM((CHUNK, D_FEAT), jnp.bfloat16) for _ in range(NB)],
        pltpu.VMEM_SHARED((N_PAD, D_FEAT), jnp.bfloat16),
        [pltpu.SemaphoreType.DMA for _ in range(NB)],
        [pltpu.SemaphoreType.DMA for _ in range(NB)],
    ],
)
def _sc_aggregate(nodes_hbm, edges_hbm, out_hbm,
                  src_v, dst_v, bufs, acc, gsems, ssems):
    c = lax.axis_index("c")
    s = lax.axis_index("s")
    wid = c * NS + s
    row0 = s * ROWS_PER_TILE

    chunk0 = wid * CHUNKS_PER_W
    pltpu.sync_copy(edges_hbm.at[0, pl.ds(chunk0, CHUNKS_PER_W)], src_v)
    pltpu.sync_copy(edges_hbm.at[1, pl.ds(chunk0, CHUNKS_PER_W)], dst_v)

    def gather(j, b):
        pltpu.async_copy(nodes_hbm.at[src_v.at[j]], bufs[b], gsems[b])

    def wait_gather(j, b):
        pltpu.make_async_copy(nodes_hbm.at[src_v.at[j]],
                              bufs[b], gsems[b]).wait()

    def scatter(j, b):
        pltpu.async_copy(bufs[b], acc.at[dst_v.at[j]], ssems[b], add=True)

    def wait_scatter(j, b):
        pltpu.make_async_copy(bufs[b], acc.at[dst_v.at[j]], ssems[b]).wait()

    for b in range(1, NB):
        gather(b, b)
    @pl.loop(0, CHUNK)
    def _zrow(r):
        for c32 in range(D_FEAT // 32):
            bufs[0][r, pl.ds(c32 * 32, 32)] = jnp.zeros((32,), jnp.bfloat16)
    for k in range(ROWS_PER_TILE // CHUNK):
        pltpu.sync_copy(bufs[0], acc.at[pl.ds(row0 + k * CHUNK, CHUNK)])
    plsc.subcore_barrier()
    gather(0, 0)

    @pl.loop(0, CHUNKS_PER_W, step=NB)
    def _(g):
        for b in range(NB):
            j = g + b
            b2 = (b + 4) % NB
            wait_gather(j, b)
            scatter(j, b)

            @pl.when((j >= 1) & (j + 4 < CHUNKS_PER_W))
            def _recycle():
                wait_scatter(j - 1, b2)
                gather(j + 4, b2)

            @pl.when((j >= 1) & (j + 4 >= CHUNKS_PER_W))
            def _drain():
                wait_scatter(j - 1, b2)

    wait_scatter(CHUNKS_PER_W - 1, (CHUNKS_PER_W - 1) % NB)

    plsc.subcore_barrier()
    pltpu.sync_copy(acc.at[pl.ds(row0, ROWS_PER_TILE)],
                    out_hbm.at[c, pl.ds(row0, ROWS_PER_TILE)])


def _combine_body(p_ref, o_ref):
    o_ref[...] = (p_ref[0].astype(jnp.float32) + p_ref[1].astype(jnp.float32))


_ROWS_BLK = 1000


def _combine(partials):
    return pl.pallas_call(
        _combine_body,
        grid=(N_NODES // _ROWS_BLK,),
        in_specs=[pl.BlockSpec((NC, _ROWS_BLK, D_FEAT), lambda i: (0, i, 0))],
        out_specs=pl.BlockSpec((_ROWS_BLK, D_FEAT), lambda i: (i, 0)),
        out_shape=jax.ShapeDtypeStruct((N_NODES, D_FEAT), jnp.float32),
    )(partials)


def kernel(node_features, edge_index):
    edges = edge_index.astype(jnp.int32).reshape(2, N_EDGES // CHUNK, CHUNK)
    nodes_bf = node_features.astype(jnp.bfloat16)
    partials = _sc_aggregate(nodes_bf, edges)
    return _combine(partials)


# speedup vs baseline: 1.5515x; 1.0332x over previous
import functools

import jax
import jax.numpy as jnp
from jax import lax
from jax.experimental import pallas as pl
from jax.experimental.pallas import tpu as pltpu
from jax.experimental.pallas import tpu_sc as plsc

N_NODES = 10000
N_EDGES = 320000
D_FEAT = 128

NC = 2
NS = 16
NW = NC * NS
CHUNK = 80
EDGES_PER_W = N_EDGES // NW
CHUNKS_PER_W = EDGES_PER_W // CHUNK
N_PAD = 10240
ROWS_PER_TILE = N_PAD // NS
NB = 5

_mesh = plsc.VectorSubcoreMesh(core_axis_name="c", subcore_axis_name="s")


@functools.partial(
    pl.kernel,
    mesh=_mesh,
    compiler_params=pltpu.CompilerParams(use_tc_tiling_on_sc=False),
    out_type=[jax.ShapeDtypeStruct((N_PAD, D_FEAT), jnp.bfloat16)
              for _ in range(NC)],
    scratch_types=[
        pltpu.VMEM((CHUNKS_PER_W, CHUNK), jnp.int32),
        pltpu.VMEM((CHUNKS_PER_W, CHUNK), jnp.int32),
        [pltpu.VMEM((CHUNK, D_FEAT), jnp.bfloat16) for _ in range(NB)],
        pltpu.VMEM_SHARED((N_PAD, D_FEAT), jnp.bfloat16),
        [pltpu.SemaphoreType.DMA for _ in range(NB)],
        [pltpu.SemaphoreType.DMA for _ in range(NB)],
    ],
)
def _sc_aggregate(nodes_hbm, edges_hbm, out0_hbm, out1_hbm,
                  src_v, dst_v, bufs, acc, gsems, ssems):
    c = lax.axis_index("c")
    s = lax.axis_index("s")
    wid = c * NS + s
    row0 = s * ROWS_PER_TILE

    chunk0 = wid * CHUNKS_PER_W
    pltpu.sync_copy(edges_hbm.at[0, pl.ds(chunk0, CHUNKS_PER_W)], src_v)
    pltpu.sync_copy(edges_hbm.at[1, pl.ds(chunk0, CHUNKS_PER_W)], dst_v)

    def gather(j, b):
        pltpu.async_copy(nodes_hbm.at[src_v.at[j]], bufs[b], gsems[b])

    def wait_gather(j, b):
        pltpu.make_async_copy(nodes_hbm.at[src_v.at[j]],
                              bufs[b], gsems[b]).wait()

    def scatter(j, b):
        pltpu.async_copy(bufs[b], acc.at[dst_v.at[j]], ssems[b], add=True)

    def wait_scatter(j, b):
        pltpu.make_async_copy(bufs[b], acc.at[dst_v.at[j]], ssems[b]).wait()

    for b in range(1, NB):
        gather(b, b)
    @pl.loop(0, CHUNK)
    def _zrow(r):
        for c32 in range(D_FEAT // 32):
            bufs[0][r, pl.ds(c32 * 32, 32)] = jnp.zeros((32,), jnp.bfloat16)
    for k in range(ROWS_PER_TILE // CHUNK):
        pltpu.sync_copy(bufs[0], acc.at[pl.ds(row0 + k * CHUNK, CHUNK)])
    plsc.subcore_barrier()
    gather(0, 0)

    @pl.loop(0, CHUNKS_PER_W, step=NB)
    def _(g):
        for b in range(NB):
            j = g + b
            b2 = (b + 4) % NB
            wait_gather(j, b)
            scatter(j, b)

            @pl.when((j >= 1) & (j + 4 < CHUNKS_PER_W))
            def _recycle():
                wait_scatter(j - 1, b2)
                gather(j + 4, b2)

            @pl.when((j >= 1) & (j + 4 >= CHUNKS_PER_W))
            def _drain():
                wait_scatter(j - 1, b2)

    wait_scatter(CHUNKS_PER_W - 1, (CHUNKS_PER_W - 1) % NB)

    plsc.subcore_barrier()

    @pl.when(c == 0)
    def _out0():
        pltpu.sync_copy(acc.at[pl.ds(row0, ROWS_PER_TILE)],
                        out0_hbm.at[pl.ds(row0, ROWS_PER_TILE)])

    @pl.when(c == 1)
    def _out1():
        pltpu.sync_copy(acc.at[pl.ds(row0, ROWS_PER_TILE)],
                        out1_hbm.at[pl.ds(row0, ROWS_PER_TILE)])


def _combine_body(p0_ref, p1_ref, o_ref):
    a = jnp.reshape(p0_ref[...], (_ROWS_BLK, D_FEAT)).astype(jnp.float32)
    b = jnp.reshape(p1_ref[...], (_ROWS_BLK, D_FEAT)).astype(jnp.float32)
    o_ref[...] = a + b


_ROWS_BLK = 1000


def _combine(p0, p1):
    return pl.pallas_call(
        _combine_body,
        grid=(N_NODES // _ROWS_BLK,),
        in_specs=[pl.BlockSpec((_ROWS_BLK * D_FEAT,), lambda i: (i,)),
                  pl.BlockSpec((_ROWS_BLK * D_FEAT,), lambda i: (i,))],
        out_specs=pl.BlockSpec((_ROWS_BLK, D_FEAT), lambda i: (i, 0)),
        out_shape=jax.ShapeDtypeStruct((N_NODES, D_FEAT), jnp.float32),
    )(p0, p1)


def kernel(node_features, edge_index):
    edges = edge_index.astype(jnp.int32).reshape(2, N_EDGES // CHUNK, CHUNK)
    nodes_bf = node_features.astype(jnp.bfloat16)
    p0, p1 = _sc_aggregate(nodes_bf, edges)
    return _combine(p0.reshape(N_PAD * D_FEAT), p1.reshape(N_PAD * D_FEAT))


# speedup vs baseline: 1.5806x; 1.0188x over previous
import functools

import jax
import jax.numpy as jnp
from jax import lax
from jax.experimental import pallas as pl
from jax.experimental.pallas import tpu as pltpu
from jax.experimental.pallas import tpu_sc as plsc

N_NODES = 10000
N_EDGES = 320000
D_FEAT = 128

NC = 2
NS = 16
NW = NC * NS
CHUNK = 40
EDGES_PER_W = N_EDGES // NW
CHUNKS_PER_W = EDGES_PER_W // CHUNK
N_PAD = 10240
ROWS_PER_TILE = N_PAD // NS
NB = 10

_mesh = plsc.VectorSubcoreMesh(core_axis_name="c", subcore_axis_name="s")


@functools.partial(
    pl.kernel,
    mesh=_mesh,
    compiler_params=pltpu.CompilerParams(use_tc_tiling_on_sc=False),
    out_type=[jax.ShapeDtypeStruct((N_PAD, D_FEAT), jnp.bfloat16)
              for _ in range(NC)],
    scratch_types=[
        pltpu.VMEM((CHUNKS_PER_W, CHUNK), jnp.int32),
        pltpu.VMEM((CHUNKS_PER_W, CHUNK), jnp.int32),
        [pltpu.VMEM((CHUNK, D_FEAT), jnp.bfloat16) for _ in range(NB)],
        pltpu.VMEM_SHARED((N_PAD, D_FEAT), jnp.bfloat16),
        [pltpu.SemaphoreType.DMA for _ in range(NB)],
        [pltpu.SemaphoreType.DMA for _ in range(NB)],
    ],
)
def _sc_aggregate(nodes_hbm, edges_hbm, out0_hbm, out1_hbm,
                  src_v, dst_v, bufs, acc, gsems, ssems):
    c = lax.axis_index("c")
    s = lax.axis_index("s")
    wid = c * NS + s
    row0 = s * ROWS_PER_TILE

    chunk0 = wid * CHUNKS_PER_W
    pltpu.sync_copy(edges_hbm.at[0, pl.ds(chunk0, CHUNKS_PER_W)], src_v)
    pltpu.sync_copy(edges_hbm.at[1, pl.ds(chunk0, CHUNKS_PER_W)], dst_v)

    def gather(j, b):
        pltpu.async_copy(nodes_hbm.at[src_v.at[j]], bufs[b], gsems[b])

    def wait_gather(j, b):
        pltpu.make_async_copy(nodes_hbm.at[src_v.at[j]],
                              bufs[b], gsems[b]).wait()

    def scatter(j, b):
        pltpu.async_copy(bufs[b], acc.at[dst_v.at[j]], ssems[b], add=True)

    def wait_scatter(j, b):
        pltpu.make_async_copy(bufs[b], acc.at[dst_v.at[j]], ssems[b]).wait()

    for b in range(1, NB):
        gather(b, b)
    @pl.loop(0, CHUNK)
    def _zrow(r):
        for c32 in range(D_FEAT // 32):
            bufs[0][r, pl.ds(c32 * 32, 32)] = jnp.zeros((32,), jnp.bfloat16)
    for k in range(ROWS_PER_TILE // CHUNK):
        pltpu.sync_copy(bufs[0], acc.at[pl.ds(row0 + k * CHUNK, CHUNK)])
    plsc.subcore_barrier()
    gather(0, 0)

    @pl.loop(0, CHUNKS_PER_W, step=NB)
    def _(g):
        for b in range(NB):
            j = g + b
            b2 = (b + NB - 1) % NB
            wait_gather(j, b)
            scatter(j, b)

            @pl.when((j >= 1) & (j + NB - 1 < CHUNKS_PER_W))
            def _recycle():
                wait_scatter(j - 1, b2)
                gather(j + NB - 1, b2)

            @pl.when((j >= 1) & (j + NB - 1 >= CHUNKS_PER_W))
            def _drain():
                wait_scatter(j - 1, b2)

    wait_scatter(CHUNKS_PER_W - 1, (CHUNKS_PER_W - 1) % NB)

    plsc.subcore_barrier()

    @pl.when(c == 0)
    def _out0():
        pltpu.sync_copy(acc.at[pl.ds(row0, ROWS_PER_TILE)],
                        out0_hbm.at[pl.ds(row0, ROWS_PER_TILE)])

    @pl.when(c == 1)
    def _out1():
        pltpu.sync_copy(acc.at[pl.ds(row0, ROWS_PER_TILE)],
                        out1_hbm.at[pl.ds(row0, ROWS_PER_TILE)])


def _combine_body(p0_ref, p1_ref, o_ref):
    a = jnp.reshape(p0_ref[...], (_ROWS_BLK, D_FEAT)).astype(jnp.float32)
    b = jnp.reshape(p1_ref[...], (_ROWS_BLK, D_FEAT)).astype(jnp.float32)
    o_ref[...] = a + b


_ROWS_BLK = 1000


def _combine(p0, p1):
    return pl.pallas_call(
        _combine_body,
        grid=(N_NODES // _ROWS_BLK,),
        in_specs=[pl.BlockSpec((_ROWS_BLK * D_FEAT,), lambda i: (i,)),
                  pl.BlockSpec((_ROWS_BLK * D_FEAT,), lambda i: (i,))],
        out_specs=pl.BlockSpec((_ROWS_BLK, D_FEAT), lambda i: (i, 0)),
        out_shape=jax.ShapeDtypeStruct((N_NODES, D_FEAT), jnp.float32),
    )(p0, p1)


def kernel(node_features, edge_index):
    edges = edge_index.astype(jnp.int32).reshape(2, N_EDGES // CHUNK, CHUNK)
    nodes_bf = node_features.astype(jnp.bfloat16)
    p0, p1 = _sc_aggregate(nodes_bf, edges)
    return _combine(p0.reshape(N_PAD * D_FEAT), p1.reshape(N_PAD * D_FEAT))
